# baseline (speedup 1.0000x reference)
import functools

import jax
import jax.numpy as jnp
from jax import lax
from jax.experimental import pallas as pl
from jax.experimental.pallas import tpu as pltpu
from jax.experimental.pallas import tpu_sc as plsc

B = 4
C = 192
IN_H = 384
IN_W = 384
OUT_H = 224
OUT_W = 224
HW = IN_H * IN_W
OHW = OUT_H * OUT_W
NPIX = B * OHW

NW = 32
TILES_PER_B = NW // B
PIX_PER_W = OHW // TILES_PER_B
CHUNK = 128
NCHUNK = PIX_PER_W // CHUNK
NGRP = CHUNK // 16
CS = C // 16
NSUB = 4
SUBROWS = CHUNK // NSUB


def _stn_body(tab, xsp, ysp, out,
              xs_v, ys_v,
              idx_a, idx_b, idx_c, idx_d,
              w_a, w_b, w_c, w_d,
              buf_a, buf_b, buf_c, buf_d, sem):
    cid = lax.axis_index("c")
    sid = lax.axis_index("s")
    wid = sid * 2 + cid
    b = wid // TILES_PER_B
    sub = wid % TILES_PER_B
    pix0 = sub * PIX_PER_W
    row_base = b * HW
    out_base = b * OHW + pix0

    pltpu.sync_copy(xsp.at[b, pl.ds(pix0, PIX_PER_W)], xs_v)
    pltpu.sync_copy(ysp.at[b, pl.ds(pix0, PIX_PER_W)], ys_v)

    def chunk_body(ci, carry):
        for g in range(NGRP):
            sl = pl.ds(g * 16, 16)
            cl = pl.ds(ci * CHUNK + g * 16, 16)
            xs = xs_v[cl]
            ys = ys_v[cl]
            x0t = xs.astype(jnp.int32)
            x0 = jnp.where(xs < x0t.astype(jnp.float32), x0t - 1, x0t)
            y0t = ys.astype(jnp.int32)
            y0 = jnp.where(ys < y0t.astype(jnp.float32), y0t - 1, y0t)
            x0c = jnp.clip(x0, 0, IN_W - 1)
            x1c = jnp.clip(x0 + 1, 0, IN_W - 1)
            y0c = jnp.clip(y0, 0, IN_H - 1)
            y1c = jnp.clip(y0 + 1, 0, IN_H - 1)
            idx_a[sl] = row_base + y0c * IN_W + x0c
            idx_b[sl] = row_base + y1c * IN_W + x0c
            idx_c[sl] = row_base + y0c * IN_W + x1c
            idx_d[sl] = row_base + y1c * IN_W + x1c
            x0f = x0c.astype(jnp.float32)
            x1f = (x0c + 1).astype(jnp.float32)
            y0f = y0c.astype(jnp.float32)
            y1f = (y0c + 1).astype(jnp.float32)
            w_a[sl] = (x1f - xs) * (y1f - ys)
            w_b[sl] = (x1f - xs) * (ys - y0f)
            w_c[sl] = (xs - x0f) * (y1f - ys)
            w_d[sl] = (xs - x0f) * (ys - y0f)

        with jax.named_scope("gather"):
            handles = []
            for q in range(NSUB):
                ql = pl.ds(q * SUBROWS, SUBROWS)
                for iref, bref in ((idx_a, buf_a), (idx_b, buf_b),
                                   (idx_c, buf_c), (idx_d, buf_d)):
                    handles.append(pltpu.async_copy(
                        tab.at[iref.at[ql]], bref.at[ql], sem))
            for h in handles:
                h.wait()

        scope = jax.named_scope("combine")
        scope.__enter__()

        @plsc.parallel_loop(0, CHUNK, unroll=4)
        def _combine(p):
            pv = jnp.full((16,), p, jnp.int32)
            wa = plsc.load_gather(w_a, [pv])
            wb = plsc.load_gather(w_b, [pv])
            wc = plsc.load_gather(w_c, [pv])
            wd = plsc.load_gather(w_d, [pv])
            for k in range(CS):
                s = pl.ds(k * 16, 16)
                v = (buf_a[p, s] * wa + buf_b[p, s] * wb
                     + buf_c[p, s] * wc + buf_d[p, s] * wd)
                buf_a[p, s] = v

        scope.__exit__(None, None, None)
        with jax.named_scope("store"):
            pltpu.sync_copy(buf_a,
                            out.at[pl.ds(out_base + ci * CHUNK, CHUNK)])
        return carry

    lax.fori_loop(0, NCHUNK, chunk_body, 0)


@jax.jit
def kernel(x, theta):
    tab = x.transpose(0, 2, 3, 1).reshape(B * HW, C)

    x_t = jnp.tile(jnp.linspace(-1.0, 1.0, OUT_W), (OUT_H, 1))
    y_t = jnp.tile(jnp.linspace(-1.0, 1.0, OUT_H).reshape(-1, 1), (1, OUT_W))
    grid = jnp.concatenate(
        [x_t.reshape(1, -1), y_t.reshape(1, -1),
         jnp.ones((1, OHW), dtype=jnp.float32)], axis=0)
    grid_b = jnp.broadcast_to(grid[None], (B, 3, OHW))
    th = theta.reshape(-1, 2, 3)
    T_g = jnp.einsum('bij,bjn->bin', th, grid_b)
    xsp = (T_g[:, 0] + 1.0) * (IN_W - 1) / 2.0
    ysp = (T_g[:, 1] + 1.0) * (IN_H - 1) / 2.0

    mesh = plsc.VectorSubcoreMesh(core_axis_name="c", subcore_axis_name="s")
    stn = functools.partial(
        pl.kernel,
        mesh=mesh,
        compiler_params=pltpu.CompilerParams(
            needs_layout_passes=False, use_tc_tiling_on_sc=False),
        out_type=jax.ShapeDtypeStruct((NPIX, C), jnp.float32),
        scratch_types=[
            pltpu.VMEM((PIX_PER_W,), jnp.float32),
            pltpu.VMEM((PIX_PER_W,), jnp.float32),
            pltpu.VMEM((CHUNK,), jnp.int32),
            pltpu.VMEM((CHUNK,), jnp.int32),
            pltpu.VMEM((CHUNK,), jnp.int32),
            pltpu.VMEM((CHUNK,), jnp.int32),
            pltpu.VMEM((CHUNK,), jnp.float32),
            pltpu.VMEM((CHUNK,), jnp.float32),
            pltpu.VMEM((CHUNK,), jnp.float32),
            pltpu.VMEM((CHUNK,), jnp.float32),
            pltpu.VMEM((CHUNK, C), jnp.float32),
            pltpu.VMEM((CHUNK, C), jnp.float32),
            pltpu.VMEM((CHUNK, C), jnp.float32),
            pltpu.VMEM((CHUNK, C), jnp.float32),
            pltpu.SemaphoreType.DMA,
        ],
    )(_stn_body)
    flat = stn(tab, xsp, ysp)
    return flat.reshape(B, OHW, C).transpose(0, 2, 1).reshape(
        B, C, OUT_H, OUT_W)


# speedup vs baseline: 1.0013x; 1.0013x over previous
import functools

import jax
import jax.numpy as jnp
from jax import lax
from jax.experimental import pallas as pl
from jax.experimental.pallas import tpu as pltpu
from jax.experimental.pallas import tpu_sc as plsc

B = 4
C = 192
IN_H = 384
IN_W = 384
OUT_H = 224
OUT_W = 224
HW = IN_H * IN_W
OHW = OUT_H * OUT_W
NPIX = B * OHW

NW = 32
TILES_PER_B = NW // B
PIX_PER_W = OHW // TILES_PER_B
CHUNK = 64
NCHUNK = PIX_PER_W // CHUNK
NGRP = CHUNK // 16
CS = C // 16
NSUB = 4
SUBROWS = CHUNK // NSUB


def _stn_body(tab, xsp, ysp, out,
              xs_v, ys_v,
              ia0, ib0, ic0, id0, ia1, ib1, ic1, id1,
              wa0, wb0, wc0, wd0, wa1, wb1, wc1, wd1,
              buf, sem0, sem1):
    cid = lax.axis_index("c")
    sid = lax.axis_index("s")
    wid = sid * 2 + cid
    b = wid // TILES_PER_B
    sub = wid % TILES_PER_B
    pix0 = sub * PIX_PER_W
    row_base = b * HW
    out_base = b * OHW + pix0

    idxs = ((ia0, ib0, ic0, id0), (ia1, ib1, ic1, id1))
    ws = ((wa0, wb0, wc0, wd0), (wa1, wb1, wc1, wd1))
    sems = (sem0, sem1)

    pltpu.sync_copy(xsp.at[b, pl.ds(pix0, PIX_PER_W)], xs_v)
    pltpu.sync_copy(ysp.at[b, pl.ds(pix0, PIX_PER_W)], ys_v)

    def work(ci, s):
        ia, ib, ic, id_ = idxs[s]
        wa_r, wb_r, wc_r, wd_r = ws[s]
        for g in range(NGRP):
            sl = pl.ds(g * 16, 16)
            cl = pl.ds(ci * CHUNK + g * 16, 16)
            xs = xs_v[cl]
            ys = ys_v[cl]
            x0t = xs.astype(jnp.int32)
            x0 = jnp.where(xs < x0t.astype(jnp.float32), x0t - 1, x0t)
            y0t = ys.astype(jnp.int32)
            y0 = jnp.where(ys < y0t.astype(jnp.float32), y0t - 1, y0t)
            x0c = jnp.clip(x0, 0, IN_W - 1)
            x1c = jnp.clip(x0 + 1, 0, IN_W - 1)
            y0c = jnp.clip(y0, 0, IN_H - 1)
            y1c = jnp.clip(y0 + 1, 0, IN_H - 1)
            ia[sl] = row_base + y0c * IN_W + x0c
            ib[sl] = row_base + y1c * IN_W + x0c
            ic[sl] = row_base + y0c * IN_W + x1c
            id_[sl] = row_base + y1c * IN_W + x1c
            x0f = x0c.astype(jnp.float32)
            x1f = (x0c + 1).astype(jnp.float32)
            y0f = y0c.astype(jnp.float32)
            y1f = (y0c + 1).astype(jnp.float32)
            wa_r[sl] = (x1f - xs) * (y1f - ys)
            wb_r[sl] = (x1f - xs) * (ys - y0f)
            wc_r[sl] = (xs - x0f) * (y1f - ys)
            wd_r[sl] = (xs - x0f) * (ys - y0f)
        for n, iref in enumerate((ia, ib, ic, id_)):
            for q in range(NSUB):
                ql = pl.ds(q * SUBROWS, SUBROWS)
                dl = pl.ds((4 * s + n) * CHUNK + q * SUBROWS, SUBROWS)
                pltpu.async_copy(tab.at[iref.at[ql]], buf.at[dl], sems[s])

    def finish(ci, s):
        ia, ib, ic, id_ = idxs[s]
        wa_r, wb_r, wc_r, wd_r = ws[s]
        for n, iref in enumerate((ia, ib, ic, id_)):
            for q in range(NSUB):
                ql = pl.ds(q * SUBROWS, SUBROWS)
                dl = pl.ds((4 * s + n) * CHUNK + q * SUBROWS, SUBROWS)
                pltpu.make_async_copy(tab.at[iref.at[ql]], buf.at[dl],
                                      sems[s]).wait()
        base = 4 * s * CHUNK

        @plsc.parallel_loop(0, CHUNK, unroll=4)
        def _combine(p):
            pv = jnp.full((16,), p, jnp.int32)
            wa = plsc.load_gather(wa_r, [pv])
            wb = plsc.load_gather(wb_r, [pv])
            wc = plsc.load_gather(wc_r, [pv])
            wd = plsc.load_gather(wd_r, [pv])
            for k in range(CS):
                sch = pl.ds(k * 16, 16)
                v = (buf[base + p, sch] * wa
                     + buf[base + CHUNK + p, sch] * wb
                     + buf[base + 2 * CHUNK + p, sch] * wc
                     + buf[base + 3 * CHUNK + p, sch] * wd)
                buf[base + p, sch] = v

        pltpu.sync_copy(buf.at[pl.ds(base, CHUNK)],
                        out.at[pl.ds(out_base + ci * CHUNK, CHUNK)])

    work(0, 0)

    def pair(k, carry):
        work(2 * k + 1, 1)
        finish(2 * k, 0)
        work(2 * k + 2, 0)
        finish(2 * k + 1, 1)
        return carry

    lax.fori_loop(0, (NCHUNK - 2) // 2, pair, 0)
    work(NCHUNK - 1, 1)
    finish(NCHUNK - 2, 0)
    finish(NCHUNK - 1, 1)


@jax.jit
def kernel(x, theta):
    tab = x.transpose(0, 2, 3, 1).reshape(B * HW, C)

    x_t = jnp.tile(jnp.linspace(-1.0, 1.0, OUT_W), (OUT_H, 1))
    y_t = jnp.tile(jnp.linspace(-1.0, 1.0, OUT_H).reshape(-1, 1), (1, OUT_W))
    grid = jnp.concatenate(
        [x_t.reshape(1, -1), y_t.reshape(1, -1),
         jnp.ones((1, OHW), dtype=jnp.float32)], axis=0)
    grid_b = jnp.broadcast_to(grid[None], (B, 3, OHW))
    th = theta.reshape(-1, 2, 3)
    T_g = jnp.einsum('bij,bjn->bin', th, grid_b)
    xsp = (T_g[:, 0] + 1.0) * (IN_W - 1) / 2.0
    ysp = (T_g[:, 1] + 1.0) * (IN_H - 1) / 2.0

    mesh = plsc.VectorSubcoreMesh(core_axis_name="c", subcore_axis_name="s")
    idx_t = pltpu.VMEM((CHUNK,), jnp.int32)
    w_t = pltpu.VMEM((CHUNK,), jnp.float32)
    stn = functools.partial(
        pl.kernel,
        mesh=mesh,
        compiler_params=pltpu.CompilerParams(
            needs_layout_passes=False, use_tc_tiling_on_sc=False),
        out_type=jax.ShapeDtypeStruct((NPIX, C), jnp.float32),
        scratch_types=[
            pltpu.VMEM((PIX_PER_W,), jnp.float32),
            pltpu.VMEM((PIX_PER_W,), jnp.float32),
            idx_t, idx_t, idx_t, idx_t,
            idx_t, idx_t, idx_t, idx_t,
            w_t, w_t, w_t, w_t,
            w_t, w_t, w_t, w_t,
            pltpu.VMEM((8 * CHUNK, C), jnp.float32),
            pltpu.SemaphoreType.DMA,
            pltpu.SemaphoreType.DMA,
        ],
    )(_stn_body)
    flat = stn(tab, xsp, ysp)
    return flat.reshape(B, OHW, C).transpose(0, 2, 1).reshape(
        B, C, OUT_H, OUT_W)
